# baseline (speedup 1.0000x reference)
import functools

import jax
import jax.numpy as jnp
from jax import lax
from jax.experimental import pallas as pl
from jax.experimental.pallas import tpu as pltpu
from jax.experimental.pallas import tpu_sc as plsc

EMB = 32
VOC = 1000000
NTILE = VOC // 128
VFULL = NTILE * 128
VPAD = VFULL + 128
NBLK = NTILE + 1


def _iota16():
    return lax.iota(jnp.int32, 16)


def _winfo():
    info = plsc.get_sparse_core_info()
    return info.num_cores, info.num_subcores


@functools.lru_cache(maxsize=None)
def _build_blocks():
    nc, ns = _winfo()
    nw = nc * ns
    nuni = (NTILE // (2 * nw)) * 2
    nuni4 = (nuni // 4) * 4
    nrem = NTILE - nuni * nw
    mesh = plsc.VectorSubcoreMesh(core_axis_name="c", subcore_axis_name="s")

    @functools.partial(
        pl.kernel,
        out_type=jax.ShapeDtypeStruct((NBLK * 32, 128), jnp.float32),
        mesh=mesh,
        scratch_types=[
            pltpu.VMEM((32, 128), jnp.float32),
            pltpu.VMEM((32, 128), jnp.float32),
            pltpu.VMEM((32, 128), jnp.float32),
            pltpu.VMEM((32, 128), jnp.float32),
            pltpu.SemaphoreType.DMA,
            pltpu.SemaphoreType.DMA,
            pltpu.SemaphoreType.DMA,
            pltpu.SemaphoreType.DMA,
            pltpu.SemaphoreType.DMA,
            pltpu.SemaphoreType.DMA,
            pltpu.SemaphoreType.DMA,
            pltpu.SemaphoreType.DMA,
        ],
        compiler_params=pltpu.CompilerParams(
            use_tc_tiling_on_sc=True, needs_layout_passes=False
        ),
    )
    def blocks(tab_hbm, tailb_hbm, blk_hbm, s0, s1, s2, s3,
               si0, si1, si2, si3, so0, so1, so2, so3):
        wid = lax.axis_index("s") * nc + lax.axis_index("c")
        stg = (s0, s1, s2, s3)
        isem = (si0, si1, si2, si3)
        osem = (so0, so1, so2, so3)

        def tile_of(j):
            return wid + j * nw

        def start_in(j, b):
            t = tile_of(j)
            v0 = pl.multiple_of(t * 128, 128)
            for p in range(4):
                pltpu.async_copy(
                    tab_hbm.at[pl.ds(p * 8, 8), pl.ds(v0, 128)],
                    stg[b].at[pl.ds(p * 8, 8)],
                    isem[b],
                )

        def wait_in(b):
            for p in range(4):
                pltpu.make_async_copy(
                    tab_hbm.at[pl.ds(0, 8), pl.ds(0, 128)],
                    stg[b].at[pl.ds(p * 8, 8)],
                    isem[b],
                ).wait()

        def start_out(j, b):
            t = tile_of(j)
            pltpu.async_copy(
                stg[b], blk_hbm.at[pl.ds(t * 32, 32)], osem[b]
            )

        def wait_out(b):
            pltpu.make_async_copy(
                stg[b], blk_hbm.at[pl.ds(0, 32)], osem[b]
            ).wait()

        for b in range(4):
            start_in(b, b)

        @pl.loop(0, nuni4, step=4)
        def _(j0):
            for b in range(4):
                j = j0 + b
                wait_in(b)
                start_out(j, b)
                wait_out(b)

                @pl.when(j + 4 < nuni4)
                def _():
                    start_in(j + 4, b)

        @pl.loop(nuni4, nuni)
        def _(j):
            start_in(j, 0)
            wait_in(0)
            pltpu.sync_copy(
                stg[0], blk_hbm.at[pl.ds(tile_of(j) * 32, 32)]
            )

        @pl.when(wid < nrem)
        def _():
            t = nuni * nw + wid
            v0 = pl.multiple_of(t * 128, 128)
            for p in range(4):
                pltpu.async_copy(
                    tab_hbm.at[pl.ds(p * 8, 8), pl.ds(v0, 128)],
                    stg[0].at[pl.ds(p * 8, 8)],
                    si0,
                )
            wait_in(0)
            pltpu.sync_copy(stg[0], blk_hbm.at[pl.ds(t * 32, 32)])

        @pl.when(wid == nrem)
        def _():
            pltpu.sync_copy(tailb_hbm, stg[0])
            pltpu.sync_copy(stg[0], blk_hbm.at[pl.ds(NTILE * 32, 32)])

    return blocks


@functools.lru_cache(maxsize=None)
def _build_transpose():
    nc, ns = _winfo()
    nw = nc * ns
    nuni = (NBLK // (2 * nw)) * 2
    nrem = NBLK - nuni * nw
    mesh = plsc.VectorSubcoreMesh(core_axis_name="c", subcore_axis_name="s")

    @functools.partial(
        pl.kernel,
        out_type=jax.ShapeDtypeStruct((VPAD, EMB), jnp.float32),
        mesh=mesh,
        scratch_types=[
            pltpu.VMEM((4096,), jnp.float32),
            pltpu.VMEM((4096,), jnp.float32),
            pltpu.VMEM((128, 33), jnp.float32),
            pltpu.VMEM((128, 33), jnp.float32),
            pltpu.SemaphoreType.DMA,
            pltpu.SemaphoreType.DMA,
            pltpu.SemaphoreType.DMA,
            pltpu.SemaphoreType.DMA,
        ],
        compiler_params=pltpu.CompilerParams(
            use_tc_tiling_on_sc=False, needs_layout_passes=False
        ),
    )
    def transp(blk_hbm, rows_hbm, s0, s1, m0, m1, si0, si1, so0, so1):
        wid = lax.axis_index("s") * nc + lax.axis_index("c")
        stg = (s0, s1)
        mid = (m0, m1)
        isem = (si0, si1)
        osem = (so0, so1)
        iota = _iota16()

        def tile_of(j):
            return wid + j * nw

        def start_in(j, b):
            t = tile_of(j)
            pltpu.async_copy(
                blk_hbm.at[pl.ds(t * 4096, 4096)], stg[b], isem[b]
            )

        def wait_in(b):
            pltpu.make_async_copy(
                blk_hbm.at[pl.ds(0, 4096)], stg[b], isem[b]
            ).wait()

        def transpose(b):
            @pl.loop(0, 128, step=16)
            def _(v0):
                vrow = iota + v0
                for e in range(EMB):
                    val = stg[b][pl.ds(e * 128 + v0, 16)]
                    col = jnp.zeros((16,), jnp.int32) + e
                    plsc.store_scatter(mid[b], [vrow, col], val)

        def start_out(j, b):
            t = tile_of(j)
            pltpu.async_copy(
                mid[b].at[:, pl.ds(0, EMB)],
                rows_hbm.at[pl.ds(t * 128, 128)],
                osem[b],
            )

        def wait_out(b):
            pltpu.make_async_copy(
                mid[b].at[:, pl.ds(0, EMB)],
                rows_hbm.at[pl.ds(0, 128)],
                osem[b],
            ).wait()

        start_in(0, 0)
        start_in(1, 1)

        @pl.loop(0, nuni, step=2)
        def _(j0):
            for b in range(2):
                j = j0 + b
                wait_in(b)

                @pl.when(j >= 2)
                def _():
                    wait_out(b)

                transpose(b)

                @pl.when(j + 2 < nuni)
                def _():
                    start_in(j + 2, b)

                start_out(j, b)

        wait_out(0)
        wait_out(1)

        @pl.when(wid < nrem)
        def _():
            t = nuni * nw + wid
            pltpu.async_copy(
                blk_hbm.at[pl.ds(t * 4096, 4096)], stg[0], si0
            )
            wait_in(0)
            transpose(0)
            pltpu.sync_copy(
                mid[0].at[:, pl.ds(0, EMB)],
                rows_hbm.at[pl.ds(t * 128, 128)],
            )

    return transp


@functools.lru_cache(maxsize=None)
def _build_gather(bsz: int, hist: int):
    n_rows = bsz * hist
    nc, ns = _winfo()
    nw = nc * ns
    rpw = n_rows // nw
    nbt = bsz // (128 * nw)
    nch = hist * nbt
    assert rpw * nw == n_rows and nbt * 128 * nw == bsz and nch % 2 == 0
    mesh = plsc.VectorSubcoreMesh(core_axis_name="c", subcore_axis_name="s")

    @functools.partial(
        pl.kernel,
        out_type=jax.ShapeDtypeStruct((hist * 4 * (bsz // 128), 8, 128),
                                      jnp.float32),
        mesh=mesh,
        scratch_types=[
            pltpu.VMEM((rpw,), jnp.int32),
            pltpu.VMEM((2, 128), jnp.int32),
            pltpu.VMEM((2, 128, EMB), jnp.float32),
            pltpu.VMEM((32, 129), jnp.float32),
            pltpu.VMEM((32, 129), jnp.float32),
            pltpu.SemaphoreType.DMA,
            pltpu.SemaphoreType.DMA,
            pltpu.SemaphoreType.DMA,
            pltpu.SemaphoreType.DMA,
        ],
        compiler_params=pltpu.CompilerParams(
            use_tc_tiling_on_sc=False, needs_layout_passes=False
        ),
    )
    def gather(idx_hbm, tab_hbm, out_hbm, idx_v, cidx_v, gbuf_v, ob0, ob1,
               sg0, sg1, so0, so1):
        obuf = (ob0, ob1)
        wid = lax.axis_index("s") * nc + lax.axis_index("c")
        gsem = (sg0, sg1)
        osem = (so0, so1)
        iota = _iota16()
        ivh = iota * hist
        erow0 = iota
        erow1 = iota + 16
        t0 = wid * nbt

        pltpu.sync_copy(idx_hbm.at[pl.ds(wid * rpw, rpw)], idx_v)

        def compact_idx(j, b):
            h = lax.shift_right_logical(j, 2)
            bt = lax.bitwise_and(j, 3)
            base = bt * (128 * hist) + h
            for bl0 in range(0, 128, 16):
                vec = ivh + (base + bl0 * hist)
                val = plsc.load_gather(idx_v, [vec])
                cidx_v[b, pl.ds(bl0, 16)] = val

        def start_gather(b):
            return pltpu.async_copy(
                tab_hbm.at[cidx_v.at[b]], gbuf_v.at[b], gsem[b]
            )

        def wait_gather(b):
            pltpu.make_async_copy(
                tab_hbm.at[cidx_v.at[b]], gbuf_v.at[b], gsem[b]
            ).wait()

        def transpose(b):
            @pl.loop(0, 128, step=4)
            def _(bl0):
                for i in range(4):
                    bl = bl0 + i
                    col = jnp.zeros((16,), jnp.int32) + bl
                    v0 = gbuf_v[b, bl, pl.ds(0, 16)]
                    v1 = gbuf_v[b, bl, pl.ds(16, 16)]
                    plsc.store_scatter(obuf[b], [erow0, col], v0)
                    plsc.store_scatter(obuf[b], [erow1, col], v1)

        def start_out(j, b):
            h = lax.shift_right_logical(j, 2)
            bt = lax.bitwise_and(j, 3)
            for e8 in range(4):
                blk = (h * 4 + e8) * (bsz // 128) + t0 + bt
                pltpu.async_copy(
                    obuf[b].at[pl.ds(e8 * 8, 8), pl.ds(0, 128)],
                    out_hbm.at[blk],
                    osem[b],
                )

        def wait_out(b):
            for e8 in range(4):
                pltpu.make_async_copy(
                    obuf[b].at[pl.ds(e8 * 8, 8), pl.ds(0, 128)],
                    out_hbm.at[0],
                    osem[b],
                ).wait()

        compact_idx(0, 0)
        start_gather(0)
        compact_idx(1, 1)
        start_gather(1)

        @pl.loop(0, nch, step=2)
        def _(j0):
            for b in range(2):
                j = j0 + b
                wait_gather(b)

                @pl.when(j >= 2)
                def _():
                    wait_out(b)

                transpose(b)
                start_out(j, b)

                @pl.when(j + 2 < nch)
                def _():
                    compact_idx(j + 2, b)
                    start_gather(b)

        wait_out(0)
        wait_out(1)

    return gather


def kernel(x, init_emb):
    bsz, hist = x.shape
    idx = x.reshape(bsz * hist).astype(jnp.int32)
    emb_t = init_emb.T
    tailb = jnp.pad(init_emb[VFULL:, :].T, ((0, 0), (0, 64)))

    blk = _build_blocks()(emb_t, tailb)
    rows = _build_transpose()(blk.reshape(NBLK * 32 * 128))
    out3 = _build_gather(bsz, hist)(idx, rows)
    out5 = out3.reshape(hist, 4, bsz // 128, 8, 128)
    return out5.transpose(2, 4, 0, 1, 3).reshape(bsz, hist, EMB)
